# baseline (speedup 1.0000x reference)
import functools

import jax
import jax.numpy as jnp
from jax import lax
from jax.experimental import pallas as pl
from jax.experimental.pallas import tpu as pltpu
from jax.experimental.pallas import tpu_sc as plsc

N = 10000
NP = 10240
E = 320000
CHUNK = 128
CPT = 80
EPT = CPT * CHUNK
EP = 32 * EPT
ER = EP // CHUNK
RPT = NP // 16

_mesh = plsc.VectorSubcoreMesh(core_axis_name="c", subcore_axis_name="s")


@functools.partial(
    pl.kernel,
    out_type=jax.ShapeDtypeStruct((2, 2, NP), jnp.float32),
    mesh=_mesh,
    scratch_types=[
        pltpu.VMEM((CPT, CHUNK), jnp.int32),
        pltpu.VMEM((CPT, CHUNK), jnp.int32),
        pltpu.VMEM((CHUNK,), jnp.float32),
        pltpu.VMEM_SHARED((NP,), jnp.float32),
        pltpu.VMEM_SHARED((NP,), jnp.float32),
    ] + [pltpu.SemaphoreType.DMA] * 4,
)
def _deg_kernel(src_h, dst_h, z1_h, out_h, sidx, didx, ones_v, acc_o, acc_i,
                sa0, sa1, sb0, sb1):
    c = lax.axis_index("c")
    s = lax.axis_index("s")
    wid = s * 2 + c
    sa = (sa0, sa1)
    sb = (sb0, sb1)
    for q in range(CHUNK // 16):
        ones_v[pl.ds(q * 16, 16)] = jnp.ones((16,), jnp.float32)
    pltpu.sync_copy(z1_h.at[pl.ds(s * RPT, RPT)], acc_o.at[pl.ds(s * RPT, RPT)])
    pltpu.sync_copy(z1_h.at[pl.ds(s * RPT, RPT)], acc_i.at[pl.ds(s * RPT, RPT)])
    pltpu.sync_copy(src_h.at[pl.ds(wid * CPT, CPT)], sidx)
    pltpu.sync_copy(dst_h.at[pl.ds(wid * CPT, CPT)], didx)
    plsc.subcore_barrier()

    def fire(j, p):
        pltpu.async_copy(ones_v, acc_o.at[sidx.at[j]], sa[p], add=True)
        pltpu.async_copy(ones_v, acc_i.at[didx.at[j]], sb[p], add=True)

    def drain(p):
        pltpu.make_async_copy(ones_v, acc_o.at[sidx.at[0]], sa[p]).wait()
        pltpu.make_async_copy(ones_v, acc_i.at[didx.at[0]], sb[p]).wait()

    fire(0, 0)
    fire(1, 1)

    def body(g, carry):
        for p in range(2):
            j = 2 * g + p
            drain(p)
            fire(j, p)
        return carry

    lax.fori_loop(1, CPT // 2, body, 0)
    drain(0)
    drain(1)
    plsc.subcore_barrier()
    pltpu.sync_copy(acc_o.at[pl.ds(s * RPT, RPT)], out_h.at[c, 0, pl.ds(s * RPT, RPT)])
    pltpu.sync_copy(acc_i.at[pl.ds(s * RPT, RPT)], out_h.at[c, 1, pl.ds(s * RPT, RPT)])


IB = 8
CPT0 = 160
NB0 = CPT0 // IB


def _make_agg(D):
    @functools.partial(
        pl.kernel,
        out_type=jax.ShapeDtypeStruct((NP, D), jnp.float32),
        mesh=_mesh,
        scratch_types=[
            pltpu.VMEM((2, IB, CHUNK), jnp.int32),
            pltpu.VMEM((2, IB, CHUNK), jnp.int32),
            pltpu.VMEM((2, CHUNK, D), jnp.float32),
            pltpu.VMEM_SHARED((NP, D), jnp.float32),
        ] + [pltpu.SemaphoreType.DMA] * 4,
    )
    def agg(src_h, dst_h, h_h, z_h, out_h, sidx, didx, rows, acc,
            ia, ib, g0, g1):
        c = lax.axis_index("c")
        s = lax.axis_index("s")
        gs = (g0, g1)

        def gfire(p, t, b):
            pltpu.async_copy(h_h.at[sidx.at[p, t]], rows.at[b], gs[b])

        def gwait(b):
            pltpu.make_async_copy(h_h.at[sidx.at[0, 0]], rows.at[b], gs[b]).wait()

        def scat(p, t, b):
            pltpu.sync_copy(rows.at[b], acc.at[didx.at[p, t]], add=True)

        def emit(base, nb):
            pltpu.sync_copy(src_h.at[pl.ds(base, IB)], sidx.at[0])
            pltpu.sync_copy(dst_h.at[pl.ds(base, IB)], didx.at[0])
            gfire(0, 0, 0)
            for B in range(nb):
                p = B % 2
                if B + 1 < nb:
                    nxt = base + (B + 1) * IB
                    pltpu.async_copy(src_h.at[pl.ds(nxt, IB)], sidx.at[1 - p], ia)
                    pltpu.async_copy(dst_h.at[pl.ds(nxt, IB)], didx.at[1 - p], ib)
                for t in range(IB - 1):
                    b = t % 2
                    gwait(b)
                    gfire(p, t + 1, 1 - b)
                    scat(p, t, b)
                gwait(1)
                scat(p, IB - 1, 1)
                if B + 1 < nb:
                    pltpu.make_async_copy(src_h.at[pl.ds(0, IB)], sidx.at[1 - p], ia).wait()
                    pltpu.make_async_copy(dst_h.at[pl.ds(0, IB)], didx.at[1 - p], ib).wait()
                    gfire(1 - p, 0, 0)

        @pl.when(c == 0)
        def _():
            pltpu.sync_copy(z_h.at[pl.ds(s * RPT, RPT)], acc.at[pl.ds(s * RPT, RPT)])
            plsc.subcore_barrier()
            emit(s * CPT0, NB0)
            plsc.subcore_barrier()
            pltpu.sync_copy(acc.at[pl.ds(s * RPT, RPT)], out_h.at[pl.ds(s * RPT, RPT)])

    return agg


_agg128 = _make_agg(128)


BR = 256
GRID = NP // BR

_col = pl.BlockSpec((BR, 1), lambda i: (i, 0))
_m128 = pl.BlockSpec((BR, 128), lambda i: (i, 0))
_m64 = pl.BlockSpec((BR, 64), lambda i: (i, 0))
_w128 = pl.BlockSpec((128, 128), lambda i: (0, 0))
_w64 = pl.BlockSpec((128, 64), lambda i: (0, 0))


def _prologue_call(doo0, doo1, dii0, dii1, x_ext):
    def body(a0, a1, b0, b1, x_ref, h_ref, ni_ref, no_ref):
        no = lax.rsqrt(jnp.maximum(a0[...] + a1[...], 1.0))
        ni = lax.rsqrt(jnp.maximum(b0[...] + b1[...], 1.0))
        h_ref[...] = x_ref[...] * no
        ni_ref[...] = ni
        no_ref[...] = no

    return pl.pallas_call(
        body,
        grid=(GRID,),
        in_specs=[_col, _col, _col, _col, _m128],
        out_specs=[_m128, _col, _col],
        out_shape=[
            jax.ShapeDtypeStruct((NP, 128), jnp.float32),
            jax.ShapeDtypeStruct((NP, 1), jnp.float32),
            jax.ShapeDtypeStruct((NP, 1), jnp.float32),
        ],
    )(doo0, doo1, dii0, dii1, x_ext)


def _mid_call(p, W, ni, no):
    def body(pr, wr, nir, nor, hr):
        h = jnp.dot(pr[...], wr[...], preferred_element_type=jnp.float32) * nir[...]
        hr[...] = jnp.maximum(h, 0.0) * nor[...]

    return pl.pallas_call(
        body,
        grid=(GRID,),
        in_specs=[_m128, _w128, _col, _col],
        out_specs=_m128,
        out_shape=jax.ShapeDtypeStruct((NP, 128), jnp.float32),
    )(p, W, ni, no)


def _final_call(p, W3, ni, b3t):
    def body(pr, w3r, nir, br, outr):
        out = jnp.dot(pr[...], w3r[...], preferred_element_type=jnp.float32)
        outr[...] = out * nir[...] + br[...]

    return pl.pallas_call(
        body,
        grid=(GRID,),
        in_specs=[_m128, _w64, _col, pl.BlockSpec((BR, 64), lambda i: (0, 0))],
        out_specs=_m64,
        out_shape=jax.ShapeDtypeStruct((NP, 64), jnp.float32),
    )(p, W3, ni, b3t)


def kernel(edge_index, x, W1, W2, W3, b3):
    src = edge_index[0].astype(jnp.int32)
    dst = edge_index[1].astype(jnp.int32)
    pad = jnp.full((EP - E,), N, jnp.int32)
    src_p = jnp.concatenate([src, pad]).reshape(ER, CHUNK)
    dst_p = jnp.concatenate([dst, pad]).reshape(ER, CHUNK)
    x_ext = jnp.zeros((NP, 128), jnp.float32).at[:N].set(x)
    z128 = jnp.zeros((NP, 128), jnp.float32)
    z1 = jnp.zeros((NP,), jnp.float32)

    deg = _deg_kernel(src_p, dst_p, z1)
    doo0 = deg[0, 0].reshape(NP, 1)
    doo1 = deg[1, 0].reshape(NP, 1)
    dii0 = deg[0, 1].reshape(NP, 1)
    dii1 = deg[1, 1].reshape(NP, 1)

    h0s, ni, no = _prologue_call(doo0, doo1, dii0, dii1, x_ext)
    p = _agg128(src_p, dst_p, h0s, z128)
    h1s = _mid_call(p, W1, ni, no)
    p = _agg128(src_p, dst_p, h1s, z128)
    h2s = _mid_call(p, W2, ni, no)
    p = _agg128(src_p, dst_p, h2s, z128)
    out = _final_call(p, W3, ni,
                      jnp.broadcast_to(b3.reshape(1, 64), (BR, 64)))
    return out[:N]


# speedup vs baseline: 1.2391x; 1.2391x over previous
import functools

import jax
import jax.numpy as jnp
from jax import lax
from jax.experimental import pallas as pl
from jax.experimental.pallas import tpu as pltpu
from jax.experimental.pallas import tpu_sc as plsc

N = 10000
NP = 10240
E = 320000
CHUNK = 128
CPT = 80
EPT = CPT * CHUNK
EP = 32 * EPT
ER = EP // CHUNK
RPT = NP // 16

_mesh = plsc.VectorSubcoreMesh(core_axis_name="c", subcore_axis_name="s")


@functools.partial(
    pl.kernel,
    out_type=jax.ShapeDtypeStruct((2, 2, NP), jnp.float32),
    mesh=_mesh,
    scratch_types=[
        pltpu.VMEM((CPT, CHUNK), jnp.int32),
        pltpu.VMEM((CPT, CHUNK), jnp.int32),
        pltpu.VMEM((CHUNK,), jnp.float32),
        pltpu.VMEM_SHARED((NP,), jnp.float32),
        pltpu.VMEM_SHARED((NP,), jnp.float32),
    ] + [pltpu.SemaphoreType.DMA] * 4,
)
def _deg_kernel(src_h, dst_h, z1_h, out_h, sidx, didx, ones_v, acc_o, acc_i,
                sa0, sa1, sb0, sb1):
    c = lax.axis_index("c")
    s = lax.axis_index("s")
    wid = s * 2 + c
    sa = (sa0, sa1)
    sb = (sb0, sb1)
    for q in range(CHUNK // 16):
        ones_v[pl.ds(q * 16, 16)] = jnp.ones((16,), jnp.float32)
    pltpu.sync_copy(z1_h.at[pl.ds(s * RPT, RPT)], acc_o.at[pl.ds(s * RPT, RPT)])
    pltpu.sync_copy(z1_h.at[pl.ds(s * RPT, RPT)], acc_i.at[pl.ds(s * RPT, RPT)])
    pltpu.sync_copy(src_h.at[pl.ds(wid * CPT, CPT)], sidx)
    pltpu.sync_copy(dst_h.at[pl.ds(wid * CPT, CPT)], didx)
    plsc.subcore_barrier()

    def fire(j, p):
        pltpu.async_copy(ones_v, acc_o.at[sidx.at[j]], sa[p], add=True)
        pltpu.async_copy(ones_v, acc_i.at[didx.at[j]], sb[p], add=True)

    def drain(p):
        pltpu.make_async_copy(ones_v, acc_o.at[sidx.at[0]], sa[p]).wait()
        pltpu.make_async_copy(ones_v, acc_i.at[didx.at[0]], sb[p]).wait()

    fire(0, 0)
    fire(1, 1)

    def body(g, carry):
        for p in range(2):
            j = 2 * g + p
            drain(p)
            fire(j, p)
        return carry

    lax.fori_loop(1, CPT // 2, body, 0)
    drain(0)
    drain(1)
    plsc.subcore_barrier()
    pltpu.sync_copy(acc_o.at[pl.ds(s * RPT, RPT)], out_h.at[c, 0, pl.ds(s * RPT, RPT)])
    pltpu.sync_copy(acc_i.at[pl.ds(s * RPT, RPT)], out_h.at[c, 1, pl.ds(s * RPT, RPT)])


IB = 8
CPT0 = 112
CPT1 = 48
NB0 = CPT0 // IB
NB1 = CPT1 // IB


def _make_agg(D):
    @functools.partial(
        pl.kernel,
        out_type=jax.ShapeDtypeStruct((2, NP, D), jnp.float32),
        mesh=_mesh,
        scratch_types=[
            pltpu.VMEM((2, IB, CHUNK), jnp.int32),
            pltpu.VMEM((2, IB, CHUNK), jnp.int32),
            pltpu.VMEM((2, CHUNK, D), jnp.float32),
            pltpu.VMEM_SHARED((NP, D), jnp.float32),
        ] + [pltpu.SemaphoreType.DMA] * 4,
    )
    def agg(src_h, dst_h, h_h, z_h, out_h, sidx, didx, rows, acc,
            ia, ib, g0, g1):
        c = lax.axis_index("c")
        s = lax.axis_index("s")
        gs = (g0, g1)
        pltpu.sync_copy(z_h.at[pl.ds(s * RPT, RPT)], acc.at[pl.ds(s * RPT, RPT)])
        plsc.subcore_barrier()

        def gfire(p, t, b):
            pltpu.async_copy(h_h.at[sidx.at[p, t]], rows.at[b], gs[b])

        def gwait(b):
            pltpu.make_async_copy(h_h.at[sidx.at[0, 0]], rows.at[b], gs[b]).wait()

        def scat(p, t, b):
            pltpu.sync_copy(rows.at[b], acc.at[didx.at[p, t]], add=True)

        def emit(base, nb):
            pltpu.sync_copy(src_h.at[pl.ds(base, IB)], sidx.at[0])
            pltpu.sync_copy(dst_h.at[pl.ds(base, IB)], didx.at[0])
            gfire(0, 0, 0)
            for B in range(nb):
                p = B % 2
                if B + 1 < nb:
                    nxt = base + (B + 1) * IB
                    pltpu.async_copy(src_h.at[pl.ds(nxt, IB)], sidx.at[1 - p], ia)
                    pltpu.async_copy(dst_h.at[pl.ds(nxt, IB)], didx.at[1 - p], ib)
                for t in range(IB - 1):
                    b = t % 2
                    gwait(b)
                    gfire(p, t + 1, 1 - b)
                    scat(p, t, b)
                gwait(1)
                scat(p, IB - 1, 1)
                if B + 1 < nb:
                    pltpu.make_async_copy(src_h.at[pl.ds(0, IB)], sidx.at[1 - p], ia).wait()
                    pltpu.make_async_copy(dst_h.at[pl.ds(0, IB)], didx.at[1 - p], ib).wait()
                    gfire(1 - p, 0, 0)

        def emit_serial(base, nb):
            for B in range(nb):
                pltpu.sync_copy(src_h.at[pl.ds(base + B * IB, IB)], sidx.at[0])
                pltpu.sync_copy(dst_h.at[pl.ds(base + B * IB, IB)], didx.at[0])
                for t in range(IB):
                    pltpu.sync_copy(h_h.at[sidx.at[0, t]], rows.at[0])
                    scat(0, t, 0)

        @pl.when(c == 0)
        def _():
            emit(s * CPT0, NB0)

        @pl.when(c == 1)
        def _():
            emit_serial(16 * CPT0 + s * CPT1, NB1)

        plsc.subcore_barrier()
        pltpu.sync_copy(acc.at[pl.ds(s * RPT, RPT)], out_h.at[c, pl.ds(s * RPT, RPT)])

    return agg


_agg128 = _make_agg(128)


BR = 256
GRID = NP // BR

_col = pl.BlockSpec((BR, 1), lambda i: (i, 0))
_m128 = pl.BlockSpec((BR, 128), lambda i: (i, 0))
_m64 = pl.BlockSpec((BR, 64), lambda i: (i, 0))
_w128 = pl.BlockSpec((128, 128), lambda i: (0, 0))
_w64 = pl.BlockSpec((128, 64), lambda i: (0, 0))


def _prologue_call(doo0, doo1, dii0, dii1, x_ext):
    def body(a0, a1, b0, b1, x_ref, h_ref, ni_ref, no_ref):
        no = lax.rsqrt(jnp.maximum(a0[...] + a1[...], 1.0))
        ni = lax.rsqrt(jnp.maximum(b0[...] + b1[...], 1.0))
        h_ref[...] = x_ref[...] * no
        ni_ref[...] = ni
        no_ref[...] = no

    return pl.pallas_call(
        body,
        grid=(GRID,),
        in_specs=[_col, _col, _col, _col, _m128],
        out_specs=[_m128, _col, _col],
        out_shape=[
            jax.ShapeDtypeStruct((NP, 128), jnp.float32),
            jax.ShapeDtypeStruct((NP, 1), jnp.float32),
            jax.ShapeDtypeStruct((NP, 1), jnp.float32),
        ],
    )(doo0, doo1, dii0, dii1, x_ext)


def _mid_call(p0, p1, W, ni, no):
    def body(p0r, p1r, wr, nir, nor, hr):
        agg = p0r[...] + p1r[...]
        h = jnp.dot(agg, wr[...], preferred_element_type=jnp.float32) * nir[...]
        hr[...] = jnp.maximum(h, 0.0) * nor[...]

    return pl.pallas_call(
        body,
        grid=(GRID,),
        in_specs=[_m128, _m128, _w128, _col, _col],
        out_specs=_m128,
        out_shape=jax.ShapeDtypeStruct((NP, 128), jnp.float32),
    )(p0, p1, W, ni, no)


def _final_call(p0, p1, W3, ni, b3t):
    def body(p0r, p1r, w3r, nir, br, outr):
        agg = p0r[...] + p1r[...]
        out = jnp.dot(agg, w3r[...], preferred_element_type=jnp.float32)
        outr[...] = out * nir[...] + br[...]

    return pl.pallas_call(
        body,
        grid=(GRID,),
        in_specs=[_m128, _m128, _w64, _col, pl.BlockSpec((BR, 64), lambda i: (0, 0))],
        out_specs=_m64,
        out_shape=jax.ShapeDtypeStruct((NP, 64), jnp.float32),
    )(p0, p1, W3, ni, b3t)


def kernel(edge_index, x, W1, W2, W3, b3):
    src = edge_index[0].astype(jnp.int32)
    dst = edge_index[1].astype(jnp.int32)
    pad = jnp.full((EP - E,), N, jnp.int32)
    src_p = jnp.concatenate([src, pad]).reshape(ER, CHUNK)
    dst_p = jnp.concatenate([dst, pad]).reshape(ER, CHUNK)
    x_ext = jnp.zeros((NP, 128), jnp.float32).at[:N].set(x)
    z128 = jnp.zeros((NP, 128), jnp.float32)
    z1 = jnp.zeros((NP,), jnp.float32)

    deg = _deg_kernel(src_p, dst_p, z1)
    doo0 = deg[0, 0].reshape(NP, 1)
    doo1 = deg[1, 0].reshape(NP, 1)
    dii0 = deg[0, 1].reshape(NP, 1)
    dii1 = deg[1, 1].reshape(NP, 1)

    h0s, ni, no = _prologue_call(doo0, doo1, dii0, dii1, x_ext)
    p = _agg128(src_p, dst_p, h0s, z128)
    h1s = _mid_call(p[0], p[1], W1, ni, no)
    p = _agg128(src_p, dst_p, h1s, z128)
    h2s = _mid_call(p[0], p[1], W2, ni, no)
    p = _agg128(src_p, dst_p, h2s, z128)
    out = _final_call(p[0], p[1], W3, ni,
                      jnp.broadcast_to(b3.reshape(1, 64), (BR, 64)))
    return out[:N]


# speedup vs baseline: 1.4554x; 1.1746x over previous
import functools

import jax
import jax.numpy as jnp
from jax import lax
from jax.experimental import pallas as pl
from jax.experimental.pallas import tpu as pltpu
from jax.experimental.pallas import tpu_sc as plsc

N = 10000
NP = 10240
E = 320000
CHUNK = 128
CPT = 80
EPT = CPT * CHUNK
EP = 32 * EPT
ER = EP // CHUNK
RPT = NP // 16

_mesh = plsc.VectorSubcoreMesh(core_axis_name="c", subcore_axis_name="s")


@functools.partial(
    pl.kernel,
    out_type=jax.ShapeDtypeStruct((2, 2, NP), jnp.float32),
    mesh=_mesh,
    scratch_types=[
        pltpu.VMEM((CPT, CHUNK), jnp.int32),
        pltpu.VMEM((CPT, CHUNK), jnp.int32),
        pltpu.VMEM((CHUNK,), jnp.float32),
        pltpu.VMEM_SHARED((NP,), jnp.float32),
        pltpu.VMEM_SHARED((NP,), jnp.float32),
    ] + [pltpu.SemaphoreType.DMA] * 4,
)
def _deg_kernel(src_h, dst_h, z1_h, out_h, sidx, didx, ones_v, acc_o, acc_i,
                sa0, sa1, sb0, sb1):
    c = lax.axis_index("c")
    s = lax.axis_index("s")
    wid = s * 2 + c
    sa = (sa0, sa1)
    sb = (sb0, sb1)
    for q in range(CHUNK // 16):
        ones_v[pl.ds(q * 16, 16)] = jnp.ones((16,), jnp.float32)
    pltpu.sync_copy(z1_h.at[pl.ds(s * RPT, RPT)], acc_o.at[pl.ds(s * RPT, RPT)])
    pltpu.sync_copy(z1_h.at[pl.ds(s * RPT, RPT)], acc_i.at[pl.ds(s * RPT, RPT)])
    pltpu.sync_copy(src_h.at[pl.ds(wid * CPT, CPT)], sidx)
    pltpu.sync_copy(dst_h.at[pl.ds(wid * CPT, CPT)], didx)
    plsc.subcore_barrier()

    def fire(j, p):
        pltpu.async_copy(ones_v, acc_o.at[sidx.at[j]], sa[p], add=True)
        pltpu.async_copy(ones_v, acc_i.at[didx.at[j]], sb[p], add=True)

    def drain(p):
        pltpu.make_async_copy(ones_v, acc_o.at[sidx.at[0]], sa[p]).wait()
        pltpu.make_async_copy(ones_v, acc_i.at[didx.at[0]], sb[p]).wait()

    fire(0, 0)
    fire(1, 1)

    def body(g, carry):
        for p in range(2):
            j = 2 * g + p
            drain(p)
            fire(j, p)
        return carry

    lax.fori_loop(1, CPT // 2, body, 0)
    drain(0)
    drain(1)
    plsc.subcore_barrier()
    pltpu.sync_copy(acc_o.at[pl.ds(s * RPT, RPT)], out_h.at[c, 0, pl.ds(s * RPT, RPT)])
    pltpu.sync_copy(acc_i.at[pl.ds(s * RPT, RPT)], out_h.at[c, 1, pl.ds(s * RPT, RPT)])


IB = 8
CPT0 = 136
CPT1 = 24
NB0 = CPT0 // IB
NB1 = CPT1 // IB


def _make_agg(D):
    @functools.partial(
        pl.kernel,
        out_type=jax.ShapeDtypeStruct((2, NP, D), jnp.float32),
        mesh=_mesh,
        scratch_types=[
            pltpu.VMEM((2, IB, CHUNK), jnp.int32),
            pltpu.VMEM((2, IB, CHUNK), jnp.int32),
            pltpu.VMEM((2, CHUNK, D), jnp.float32),
            pltpu.VMEM((8, D), jnp.float32),
            pltpu.VMEM_SHARED((NP, D), jnp.float32),
        ] + [pltpu.SemaphoreType.DMA] * 4,
    )
    def agg(src_h, dst_h, h_h, out_h, sidx, didx, rows, zbuf, acc,
            ia, ib, g0, g1):
        c = lax.axis_index("c")
        s = lax.axis_index("s")
        gs = (g0, g1)
        for i in range(8):
            for q in range(D // 16):
                zbuf[i, pl.ds(q * 16, 16)] = jnp.zeros((16,), jnp.float32)

        def zrow(k, carry):
            pltpu.sync_copy(zbuf, acc.at[pl.ds(s * RPT + k * 8, 8)])
            return carry

        lax.fori_loop(0, RPT // 8, zrow, 0)
        plsc.subcore_barrier()

        def gfire(p, t, b):
            pltpu.async_copy(h_h.at[sidx.at[p, t]], rows.at[b], gs[b])

        def gwait(b):
            pltpu.make_async_copy(h_h.at[sidx.at[0, 0]], rows.at[b], gs[b]).wait()

        def scat(p, t, b):
            pltpu.sync_copy(rows.at[b], acc.at[didx.at[p, t]], add=True)

        def emit(base, nb):
            pltpu.sync_copy(src_h.at[pl.ds(base, IB)], sidx.at[0])
            pltpu.sync_copy(dst_h.at[pl.ds(base, IB)], didx.at[0])
            gfire(0, 0, 0)
            for B in range(nb):
                p = B % 2
                if B + 1 < nb:
                    nxt = base + (B + 1) * IB
                    pltpu.async_copy(src_h.at[pl.ds(nxt, IB)], sidx.at[1 - p], ia)
                    pltpu.async_copy(dst_h.at[pl.ds(nxt, IB)], didx.at[1 - p], ib)
                for t in range(IB - 1):
                    b = t % 2
                    gwait(b)
                    gfire(p, t + 1, 1 - b)
                    scat(p, t, b)
                gwait(1)
                scat(p, IB - 1, 1)
                if B + 1 < nb:
                    pltpu.make_async_copy(src_h.at[pl.ds(0, IB)], sidx.at[1 - p], ia).wait()
                    pltpu.make_async_copy(dst_h.at[pl.ds(0, IB)], didx.at[1 - p], ib).wait()
                    gfire(1 - p, 0, 0)

        def emit_serial(base, nb):
            for B in range(nb):
                pltpu.sync_copy(src_h.at[pl.ds(base + B * IB, IB)], sidx.at[0])
                pltpu.sync_copy(dst_h.at[pl.ds(base + B * IB, IB)], didx.at[0])
                for t in range(IB):
                    pltpu.sync_copy(h_h.at[sidx.at[0, t]], rows.at[0])
                    scat(0, t, 0)

        @pl.when(c == 0)
        def _():
            emit(s * CPT0, NB0)

        @pl.when(c == 1)
        def _():
            emit_serial(16 * CPT0 + s * CPT1, NB1)

        plsc.subcore_barrier()
        pltpu.sync_copy(acc.at[pl.ds(s * RPT, RPT)], out_h.at[c, pl.ds(s * RPT, RPT)])

    return agg


_agg128 = _make_agg(128)


BR = 256
GRID = NP // BR

_col = pl.BlockSpec((BR, 1), lambda i: (i, 0))
_m128 = pl.BlockSpec((BR, 128), lambda i: (i, 0))
_m64 = pl.BlockSpec((BR, 64), lambda i: (i, 0))
_w128 = pl.BlockSpec((128, 128), lambda i: (0, 0))
_w64 = pl.BlockSpec((128, 64), lambda i: (0, 0))


def _prologue_call(doo0, doo1, dii0, dii1, x_ext):
    def body(a0, a1, b0, b1, x_ref, h_ref, ni_ref, no_ref):
        no = lax.rsqrt(jnp.maximum(a0[...] + a1[...], 1.0))
        ni = lax.rsqrt(jnp.maximum(b0[...] + b1[...], 1.0))
        h_ref[...] = x_ref[...] * no
        ni_ref[...] = ni
        no_ref[...] = no

    return pl.pallas_call(
        body,
        grid=(GRID,),
        in_specs=[_col, _col, _col, _col, _m128],
        out_specs=[_m128, _col, _col],
        out_shape=[
            jax.ShapeDtypeStruct((NP, 128), jnp.float32),
            jax.ShapeDtypeStruct((NP, 1), jnp.float32),
            jax.ShapeDtypeStruct((NP, 1), jnp.float32),
        ],
    )(doo0, doo1, dii0, dii1, x_ext)


def _mid_call(p0, p1, W, ni, no):
    def body(p0r, p1r, wr, nir, nor, hr):
        agg = p0r[...] + p1r[...]
        h = jnp.dot(agg, wr[...], preferred_element_type=jnp.float32) * nir[...]
        hr[...] = jnp.maximum(h, 0.0) * nor[...]

    return pl.pallas_call(
        body,
        grid=(GRID,),
        in_specs=[_m128, _m128, _w128, _col, _col],
        out_specs=_m128,
        out_shape=jax.ShapeDtypeStruct((NP, 128), jnp.float32),
    )(p0, p1, W, ni, no)


def _final_call(p0, p1, W3, ni, b3t):
    def body(p0r, p1r, w3r, nir, br, outr):
        agg = p0r[...] + p1r[...]
        out = jnp.dot(agg, w3r[...], preferred_element_type=jnp.float32)
        outr[...] = out * nir[...] + br[...]

    return pl.pallas_call(
        body,
        grid=(GRID,),
        in_specs=[_m128, _m128, _w64, _col, pl.BlockSpec((BR, 64), lambda i: (0, 0))],
        out_specs=_m64,
        out_shape=jax.ShapeDtypeStruct((NP, 64), jnp.float32),
    )(p0, p1, W3, ni, b3t)


def kernel(edge_index, x, W1, W2, W3, b3):
    src = edge_index[0].astype(jnp.int32)
    dst = edge_index[1].astype(jnp.int32)
    pad = jnp.full((EP - E,), N, jnp.int32)
    src_p = jnp.concatenate([src, pad]).reshape(ER, CHUNK)
    dst_p = jnp.concatenate([dst, pad]).reshape(ER, CHUNK)
    x_ext = jnp.zeros((NP, 128), jnp.float32).at[:N].set(x)
    z1 = jnp.zeros((NP,), jnp.float32)

    deg = _deg_kernel(src_p, dst_p, z1)
    doo0 = deg[0, 0].reshape(NP, 1)
    doo1 = deg[1, 0].reshape(NP, 1)
    dii0 = deg[0, 1].reshape(NP, 1)
    dii1 = deg[1, 1].reshape(NP, 1)

    h0s, ni, no = _prologue_call(doo0, doo1, dii0, dii1, x_ext)
    p = _agg128(src_p, dst_p, h0s)
    h1s = _mid_call(p[0], p[1], W1, ni, no)
    p = _agg128(src_p, dst_p, h1s)
    h2s = _mid_call(p[0], p[1], W2, ni, no)
    p = _agg128(src_p, dst_p, h2s)
    out = _final_call(p[0], p[1], W3, ni,
                      jnp.broadcast_to(b3.reshape(1, 64), (BR, 64)))
    return out[:N]


# speedup vs baseline: 1.4939x; 1.0265x over previous
import functools

import jax
import jax.numpy as jnp
from jax import lax
from jax.experimental import pallas as pl
from jax.experimental.pallas import tpu as pltpu
from jax.experimental.pallas import tpu_sc as plsc

N = 10000
NP = 10240
E = 320000
CHUNK = 128
CPT = 80
EPT = CPT * CHUNK
EP = 32 * EPT
ER = EP // CHUNK
RPT = NP // 16

_mesh = plsc.VectorSubcoreMesh(core_axis_name="c", subcore_axis_name="s")


@functools.partial(
    pl.kernel,
    out_type=jax.ShapeDtypeStruct((2, 2, NP), jnp.float32),
    mesh=_mesh,
    scratch_types=[
        pltpu.VMEM((CPT, CHUNK), jnp.int32),
        pltpu.VMEM((CPT, CHUNK), jnp.int32),
        pltpu.VMEM((CHUNK,), jnp.float32),
        pltpu.VMEM_SHARED((NP,), jnp.float32),
        pltpu.VMEM_SHARED((NP,), jnp.float32),
    ] + [pltpu.SemaphoreType.DMA] * 4,
)
def _deg_kernel(src_h, dst_h, z1_h, out_h, sidx, didx, ones_v, acc_o, acc_i,
                sa0, sa1, sb0, sb1):
    c = lax.axis_index("c")
    s = lax.axis_index("s")
    wid = s * 2 + c
    sa = (sa0, sa1)
    sb = (sb0, sb1)
    for q in range(CHUNK // 16):
        ones_v[pl.ds(q * 16, 16)] = jnp.ones((16,), jnp.float32)
    pltpu.sync_copy(z1_h.at[pl.ds(s * RPT, RPT)], acc_o.at[pl.ds(s * RPT, RPT)])
    pltpu.sync_copy(z1_h.at[pl.ds(s * RPT, RPT)], acc_i.at[pl.ds(s * RPT, RPT)])
    pltpu.sync_copy(src_h.at[pl.ds(wid * CPT, CPT)], sidx)
    pltpu.sync_copy(dst_h.at[pl.ds(wid * CPT, CPT)], didx)
    plsc.subcore_barrier()

    def fire(j, p):
        pltpu.async_copy(ones_v, acc_o.at[sidx.at[j]], sa[p], add=True)
        pltpu.async_copy(ones_v, acc_i.at[didx.at[j]], sb[p], add=True)

    def drain(p):
        pltpu.make_async_copy(ones_v, acc_o.at[sidx.at[0]], sa[p]).wait()
        pltpu.make_async_copy(ones_v, acc_i.at[didx.at[0]], sb[p]).wait()

    fire(0, 0)
    fire(1, 1)

    def body(g, carry):
        for p in range(2):
            j = 2 * g + p
            drain(p)
            fire(j, p)
        return carry

    lax.fori_loop(1, CPT // 2, body, 0)
    drain(0)
    drain(1)
    plsc.subcore_barrier()
    pltpu.sync_copy(acc_o.at[pl.ds(s * RPT, RPT)], out_h.at[c, 0, pl.ds(s * RPT, RPT)])
    pltpu.sync_copy(acc_i.at[pl.ds(s * RPT, RPT)], out_h.at[c, 1, pl.ds(s * RPT, RPT)])


IB = 8
CPT0 = 144
CPT1 = 16
NB0 = CPT0 // IB
NB1 = CPT1 // IB


def _make_agg(D):
    @functools.partial(
        pl.kernel,
        out_type=jax.ShapeDtypeStruct((2, NP, D), jnp.float32),
        mesh=_mesh,
        scratch_types=[
            pltpu.VMEM((2, IB, CHUNK), jnp.int32),
            pltpu.VMEM((2, IB, CHUNK), jnp.int32),
            pltpu.VMEM((2, CHUNK, D), jnp.float32),
            pltpu.VMEM((8, D), jnp.float32),
            pltpu.VMEM_SHARED((NP, D), jnp.float32),
        ] + [pltpu.SemaphoreType.DMA] * 4,
    )
    def agg(src_h, dst_h, h_h, out_h, sidx, didx, rows, zbuf, acc,
            ia, ib, g0, g1):
        c = lax.axis_index("c")
        s = lax.axis_index("s")
        gs = (g0, g1)
        for i in range(8):
            for q in range(D // 16):
                zbuf[i, pl.ds(q * 16, 16)] = jnp.zeros((16,), jnp.float32)

        def zrow(k, carry):
            pltpu.sync_copy(zbuf, acc.at[pl.ds(s * RPT + k * 8, 8)])
            return carry

        lax.fori_loop(0, RPT // 8, zrow, 0)
        plsc.subcore_barrier()

        def gfire(p, t, b):
            pltpu.async_copy(h_h.at[sidx.at[p, t]], rows.at[b], gs[b])

        def gwait(b):
            pltpu.make_async_copy(h_h.at[sidx.at[0, 0]], rows.at[b], gs[b]).wait()

        def scat(p, t, b):
            pltpu.sync_copy(rows.at[b], acc.at[didx.at[p, t]], add=True)

        def emit(base, nb):
            pltpu.sync_copy(src_h.at[pl.ds(base, IB)], sidx.at[0])
            pltpu.sync_copy(dst_h.at[pl.ds(base, IB)], didx.at[0])
            gfire(0, 0, 0)
            for B in range(nb):
                p = B % 2
                if B + 1 < nb:
                    nxt = base + (B + 1) * IB
                    pltpu.async_copy(src_h.at[pl.ds(nxt, IB)], sidx.at[1 - p], ia)
                    pltpu.async_copy(dst_h.at[pl.ds(nxt, IB)], didx.at[1 - p], ib)
                for t in range(IB - 1):
                    b = t % 2
                    gwait(b)
                    gfire(p, t + 1, 1 - b)
                    scat(p, t, b)
                gwait(1)
                scat(p, IB - 1, 1)
                if B + 1 < nb:
                    pltpu.make_async_copy(src_h.at[pl.ds(0, IB)], sidx.at[1 - p], ia).wait()
                    pltpu.make_async_copy(dst_h.at[pl.ds(0, IB)], didx.at[1 - p], ib).wait()
                    gfire(1 - p, 0, 0)

        def emit_serial(base, nb):
            for B in range(nb):
                pltpu.sync_copy(src_h.at[pl.ds(base + B * IB, IB)], sidx.at[0])
                pltpu.sync_copy(dst_h.at[pl.ds(base + B * IB, IB)], didx.at[0])
                for t in range(IB):
                    pltpu.sync_copy(h_h.at[sidx.at[0, t]], rows.at[0])
                    scat(0, t, 0)

        @pl.when(c == 0)
        def _():
            emit(s * CPT0, NB0)

        @pl.when(c == 1)
        def _():
            emit_serial(16 * CPT0 + s * CPT1, NB1)

        plsc.subcore_barrier()
        pltpu.sync_copy(acc.at[pl.ds(s * RPT, RPT)], out_h.at[c, pl.ds(s * RPT, RPT)])

    return agg


_agg128 = _make_agg(128)


BR = 256
GRID = NP // BR

_col = pl.BlockSpec((BR, 1), lambda i: (i, 0))
_m128 = pl.BlockSpec((BR, 128), lambda i: (i, 0))
_m64 = pl.BlockSpec((BR, 64), lambda i: (i, 0))
_w128 = pl.BlockSpec((128, 128), lambda i: (0, 0))
_w64 = pl.BlockSpec((128, 64), lambda i: (0, 0))


def _prologue_call(doo0, doo1, dii0, dii1, x_ext):
    def body(a0, a1, b0, b1, x_ref, h_ref, ni_ref, no_ref):
        no = lax.rsqrt(jnp.maximum(a0[...] + a1[...], 1.0))
        ni = lax.rsqrt(jnp.maximum(b0[...] + b1[...], 1.0))
        h_ref[...] = x_ref[...] * no
        ni_ref[...] = ni
        no_ref[...] = no

    return pl.pallas_call(
        body,
        grid=(GRID,),
        in_specs=[_col, _col, _col, _col, _m128],
        out_specs=[_m128, _col, _col],
        out_shape=[
            jax.ShapeDtypeStruct((NP, 128), jnp.float32),
            jax.ShapeDtypeStruct((NP, 1), jnp.float32),
            jax.ShapeDtypeStruct((NP, 1), jnp.float32),
        ],
    )(doo0, doo1, dii0, dii1, x_ext)


def _mid_call(p0, p1, W, ni, no):
    def body(p0r, p1r, wr, nir, nor, hr):
        agg = p0r[...] + p1r[...]
        h = jnp.dot(agg, wr[...], preferred_element_type=jnp.float32) * nir[...]
        hr[...] = jnp.maximum(h, 0.0) * nor[...]

    return pl.pallas_call(
        body,
        grid=(GRID,),
        in_specs=[_m128, _m128, _w128, _col, _col],
        out_specs=_m128,
        out_shape=jax.ShapeDtypeStruct((NP, 128), jnp.float32),
    )(p0, p1, W, ni, no)


def _final_call(p0, p1, W3, ni, b3t):
    def body(p0r, p1r, w3r, nir, br, outr):
        agg = p0r[...] + p1r[...]
        out = jnp.dot(agg, w3r[...], preferred_element_type=jnp.float32)
        outr[...] = out * nir[...] + br[...]

    return pl.pallas_call(
        body,
        grid=(GRID,),
        in_specs=[_m128, _m128, _w64, _col, pl.BlockSpec((BR, 64), lambda i: (0, 0))],
        out_specs=_m64,
        out_shape=jax.ShapeDtypeStruct((NP, 64), jnp.float32),
    )(p0, p1, W3, ni, b3t)


def kernel(edge_index, x, W1, W2, W3, b3):
    src = edge_index[0].astype(jnp.int32)
    dst = edge_index[1].astype(jnp.int32)
    pad = jnp.full((EP - E,), N, jnp.int32)
    src_p = jnp.concatenate([src, pad]).reshape(ER, CHUNK)
    dst_p = jnp.concatenate([dst, pad]).reshape(ER, CHUNK)
    x_ext = jnp.zeros((NP, 128), jnp.float32).at[:N].set(x)
    z1 = jnp.zeros((NP,), jnp.float32)

    deg = _deg_kernel(src_p, dst_p, z1)
    doo0 = deg[0, 0].reshape(NP, 1)
    doo1 = deg[1, 0].reshape(NP, 1)
    dii0 = deg[0, 1].reshape(NP, 1)
    dii1 = deg[1, 1].reshape(NP, 1)

    h0s, ni, no = _prologue_call(doo0, doo1, dii0, dii1, x_ext)
    p = _agg128(src_p, dst_p, h0s)
    h1s = _mid_call(p[0], p[1], W1, ni, no)
    p = _agg128(src_p, dst_p, h1s)
    h2s = _mid_call(p[0], p[1], W2, ni, no)
    p = _agg128(src_p, dst_p, h2s)
    out = _final_call(p[0], p[1], W3, ni,
                      jnp.broadcast_to(b3.reshape(1, 64), (BR, 64)))
    return out[:N]
